# baseline (speedup 1.0000x reference)
import jax
import jax.numpy as jnp
from jax import lax
from jax.experimental import pallas as pl
from jax.experimental.pallas import tpu as pltpu
from jax.experimental.pallas import tpu_sc as plsc

B = 4096
L = 50
E = 64
W = 5
M = B * L
NC = 2
NS = 16
NW = NC * NS
LANES = 16
IW = 128
T = 128
NR = (T * W) // IW
VOCAB = 100000
WP = 8
HV = VOCAB // 2
SHR = HV + 16
ZROW = HV

PER_WA = M // NW
NTA = PER_WA // T
PER_SB = M // NS
NTB = PER_SB // T


def _phase_a(sem_hbm, s2w_hbm, widx_hbm, recip_hbm, sem_v, words_v, wfl_v,
             recip_v, map_sp):
    cid = lax.axis_index("c")
    sid = lax.axis_index("s")
    wid = sid * NC + cid
    base0 = wid * PER_WA

    rows_pt = VOCAB // NS
    pltpu.sync_copy(
        s2w_hbm.at[pl.ds(sid * rows_pt, rows_pt)],
        map_sp.at[pl.ds(sid * rows_pt, rows_pt)],
    )
    plsc.subcore_barrier()

    zi16 = jnp.full((LANES,), 0, jnp.int32)
    hv16 = jnp.full((LANES,), HV, jnp.int32)
    zrow16 = jnp.full((LANES,), ZROW, jnp.int32)
    hvm1 = jnp.full((LANES,), HV - 1, jnp.int32)
    wv = jnp.full((LANES,), W, jnp.int32)

    def tile(g, carry):
        base = base0 + g * T
        pltpu.sync_copy(sem_hbm.at[pl.ds(base, T)], sem_v.at[0])
        pltpu.sync_copy(map_sp.at[sem_v.at[0]], words_v)

        for r in range(NR):
            def flat(k2, c2):
                p = lax.iota(jnp.int32, 16) + jnp.full(
                    (LANES,), r * IW + k2 * LANES, jnp.int32
                )
                items = lax.div(p, wv)
                j = p - items * wv
                w = plsc.load_gather(words_v, [items, j])
                lid0 = jnp.where(w < hv16, w, zrow16)
                lid1 = jnp.where(w >= hv16, w - hvm1, zi16)
                wfl_v[0, r, pl.ds(k2 * LANES, LANES)] = lid0
                wfl_v[1, r, pl.ds(k2 * LANES, LANES)] = lid1
                return c2

            lax.fori_loop(0, IW // LANES, flat, 0, unroll=False)

        def grp(i, c2):
            rows = lax.iota(jnp.int32, 16) + jnp.full((LANES,), i * LANES, jnp.int32)
            ones = jnp.full((LANES,), 1.0, jnp.float32)
            zeros = jnp.full((LANES,), 0.0, jnp.float32)
            cnt = zeros
            for j in range(W):
                col = jnp.full((LANES,), j, jnp.int32)
                w = plsc.load_gather(words_v, [rows, col])
                cnt = cnt + jnp.where(w != zi16, ones, zeros)
            eps = jnp.full((LANES,), 1e-6, jnp.float32)
            recip_v[pl.ds(i * LANES, LANES)] = ones / (cnt + eps)
            return c2

        lax.fori_loop(0, T // LANES, grp, 0, unroll=False)

        rbase = (base // IW) * W
        for c in range(NC):
            pltpu.sync_copy(wfl_v.at[c], widx_hbm.at[c, pl.ds(rbase, NR)])
        pltpu.sync_copy(recip_v, recip_hbm.at[pl.ds(base, T)])
        return carry

    lax.fori_loop(0, NTA, tile, 0, unroll=False)


def _phase_b(widx_hbm, recip_hbm, sh_hbm, out_hbm, wflat_v, gath_v, outs_v,
             recip_v, shard_sp, dsem):
    cid = lax.axis_index("c")
    sid = lax.axis_index("s")
    base0 = sid * PER_SB

    shr_pt = SHR // NS
    pltpu.sync_copy(
        sh_hbm.at[cid, pl.ds(sid * shr_pt, shr_pt)],
        shard_sp.at[pl.ds(sid * shr_pt, shr_pt)],
    )
    plsc.subcore_barrier()

    one16 = jnp.full((LANES,), 1, jnp.int32)
    sixteen = jnp.full((LANES,), 16, jnp.int32)
    himsk = jnp.full((LANES,), -65536, jnp.int32)
    ev_cols = jnp.full((LANES,), 2, jnp.int32) * lax.iota(jnp.int32, 16)

    def tile(g, carry):
        base = base0 + g * T
        rbase = (base // IW) * W
        pltpu.sync_copy(widx_hbm.at[cid, pl.ds(rbase, NR)], wflat_v)
        pltpu.sync_copy(recip_hbm.at[pl.ds(base, T)], recip_v)

        handles = [
            pltpu.async_copy(
                shard_sp.at[wflat_v.at[r]], gath_v.at[pl.ds(r * IW, IW)], dsem
            )
            for r in range(NR)
        ]
        for h in handles:
            h.wait()

        def item(t, c2):
            r = plsc.load_gather(recip_v, [jnp.full((LANES,), t, jnp.int32)])
            trow = jnp.full((LANES,), t, jnp.int32)
            for ch in range(2):
                acc = gath_v[t * W, pl.ds(ch * 32, 32)]
                for j in range(1, W):
                    acc = acc + gath_v[t * W + j, pl.ds(ch * 32, 32)]
                v = plsc.bitcast(acc, jnp.int32)
                even = plsc.bitcast(lax.shift_left(v, sixteen), jnp.float32) * r
                odd = plsc.bitcast(v & himsk, jnp.float32) * r
                base_c = jnp.full((LANES,), ch * 32, jnp.int32)
                plsc.store_scatter(outs_v, [trow, base_c + ev_cols], even)
                plsc.store_scatter(outs_v, [trow, base_c + ev_cols + one16], odd)
            return c2

        lax.fori_loop(0, T, item, 0, unroll=False)

        pltpu.sync_copy(outs_v, out_hbm.at[cid, pl.ds(base, T)])
        return carry

    lax.fori_loop(0, NTB, tile, 0, unroll=False)


def _combine(p_ref, o_ref):
    o_ref[...] = p_ref[0] + p_ref[1]


@jax.jit
def kernel(sememes, sememe_to_word, word_table):
    sem_flat = sememes.reshape(M)
    s2w_pad = jnp.concatenate(
        [sememe_to_word, jnp.zeros((VOCAB, WP - W), jnp.int32)], axis=1
    )
    row_ids = lax.broadcasted_iota(jnp.int32, (VOCAB, 1), 0)
    wtb = (word_table * (row_ids != 0)).astype(jnp.bfloat16)
    zpad = jnp.zeros((SHR - HV, E), jnp.bfloat16)
    sh0 = jnp.concatenate([wtb[:HV], zpad], axis=0)
    sh1 = jnp.concatenate(
        [jnp.zeros((1, E), jnp.bfloat16), wtb[HV:], zpad[1:]], axis=0
    )
    sh = jnp.stack([sh0, sh1])

    mesh = plsc.VectorSubcoreMesh(core_axis_name="c", subcore_axis_name="s")
    sc_params = pltpu.CompilerParams(
        needs_layout_passes=False, use_tc_tiling_on_sc=False
    )

    fa = pl.kernel(
        _phase_a,
        out_type=(
            jax.ShapeDtypeStruct((NC, M * W // IW, IW), jnp.int32),
            jax.ShapeDtypeStruct((M,), jnp.float32),
        ),
        scratch_types=[
            pltpu.VMEM((1, IW), jnp.int32),
            pltpu.VMEM((T, WP), jnp.int32),
            pltpu.VMEM((NC, NR, IW), jnp.int32),
            pltpu.VMEM((T,), jnp.float32),
            pltpu.VMEM_SHARED((VOCAB, WP), jnp.int32),
        ],
        mesh=mesh,
        compiler_params=sc_params,
    )
    widx, recip = fa(sem_flat, s2w_pad)

    fb = pl.kernel(
        _phase_b,
        out_type=jax.ShapeDtypeStruct((NC, M, E), jnp.float32),
        scratch_types=[
            pltpu.VMEM((NR, IW), jnp.int32),
            pltpu.VMEM((T * W, E), jnp.bfloat16),
            pltpu.VMEM((T, E), jnp.float32),
            pltpu.VMEM((T,), jnp.float32),
            pltpu.VMEM_SHARED((SHR, E), jnp.bfloat16),
            pltpu.SemaphoreType.DMA,
        ],
        mesh=mesh,
        compiler_params=sc_params,
    )
    partials = fb(widx, recip, sh)

    RB = 2048
    out = pl.pallas_call(
        _combine,
        out_shape=jax.ShapeDtypeStruct((M, E), jnp.float32),
        grid=(M // RB,),
        in_specs=[pl.BlockSpec((NC, RB, E), lambda i: (0, i, 0))],
        out_specs=pl.BlockSpec((RB, E), lambda i: (i, 0)),
    )(partials)
    return out.reshape(B, L, E)


# speedup vs baseline: 1.0126x; 1.0126x over previous
import jax
import jax.numpy as jnp
from jax import lax
from jax.experimental import pallas as pl
from jax.experimental.pallas import tpu as pltpu
from jax.experimental.pallas import tpu_sc as plsc

B = 4096
L = 50
E = 64
W = 5
M = B * L
NC = 2
NS = 16
NW = NC * NS
LANES = 16
IW = 128
T = 128
NR = (T * W) // IW
VOCAB = 100000
WP = 8
HV = VOCAB // 2
SHR = HV + 16
ZROW = HV

PER_WA = M // NW
NTA = PER_WA // T
PER_SB = M // NS
NTB = PER_SB // T


def _phase_a(sem_hbm, s2w_hbm, widx_hbm, recip_hbm, sem_v, words_v, wfl_v,
             recip_v, map_sp):
    cid = lax.axis_index("c")
    sid = lax.axis_index("s")
    wid = sid * NC + cid
    base0 = wid * PER_WA

    rows_pt = VOCAB // NS
    pltpu.sync_copy(
        s2w_hbm.at[pl.ds(sid * rows_pt, rows_pt)],
        map_sp.at[pl.ds(sid * rows_pt, rows_pt)],
    )
    plsc.subcore_barrier()

    zi16 = jnp.full((LANES,), 0, jnp.int32)
    hv16 = jnp.full((LANES,), HV, jnp.int32)
    zrow16 = jnp.full((LANES,), ZROW, jnp.int32)
    hvm1 = jnp.full((LANES,), HV - 1, jnp.int32)
    wv = jnp.full((LANES,), W, jnp.int32)

    def tile(g, carry):
        base = base0 + g * T
        pltpu.sync_copy(sem_hbm.at[pl.ds(base, T)], sem_v.at[0])
        pltpu.sync_copy(map_sp.at[sem_v.at[0]], words_v)

        for r in range(NR):
            def flat(k2, c2):
                p = lax.iota(jnp.int32, 16) + jnp.full(
                    (LANES,), r * IW + k2 * LANES, jnp.int32
                )
                items = lax.div(p, wv)
                j = p - items * wv
                w = plsc.load_gather(words_v, [items, j])
                lid0 = jnp.where(w < hv16, w, zrow16)
                lid1 = jnp.where(w >= hv16, w - hvm1, zi16)
                wfl_v[0, r, pl.ds(k2 * LANES, LANES)] = lid0
                wfl_v[1, r, pl.ds(k2 * LANES, LANES)] = lid1
                return c2

            lax.fori_loop(0, IW // LANES, flat, 0, unroll=False)

        def grp(i, c2):
            rows = lax.iota(jnp.int32, 16) + jnp.full((LANES,), i * LANES, jnp.int32)
            ones = jnp.full((LANES,), 1.0, jnp.float32)
            zeros = jnp.full((LANES,), 0.0, jnp.float32)
            cnt = zeros
            for j in range(W):
                col = jnp.full((LANES,), j, jnp.int32)
                w = plsc.load_gather(words_v, [rows, col])
                cnt = cnt + jnp.where(w != zi16, ones, zeros)
            eps = jnp.full((LANES,), 1e-6, jnp.float32)
            recip_v[pl.ds(i * LANES, LANES)] = ones / (cnt + eps)
            return c2

        lax.fori_loop(0, T // LANES, grp, 0, unroll=False)

        rbase = (base // IW) * W
        for c in range(NC):
            pltpu.sync_copy(wfl_v.at[c], widx_hbm.at[c, pl.ds(rbase, NR)])
        pltpu.sync_copy(recip_v, recip_hbm.at[pl.ds(base, T)])
        return carry

    lax.fori_loop(0, NTA, tile, 0, unroll=False)


def _phase_b(widx_hbm, recip_hbm, sh_hbm, out_hbm, wflat_v, gath_v, outs_v,
             recip_v, shard_sp, dsem):
    cid = lax.axis_index("c")
    sid = lax.axis_index("s")
    base0 = sid * PER_SB

    shr_pt = SHR // NS
    pltpu.sync_copy(
        sh_hbm.at[cid, pl.ds(sid * shr_pt, shr_pt)],
        shard_sp.at[pl.ds(sid * shr_pt, shr_pt)],
    )
    plsc.subcore_barrier()

    one16 = jnp.full((LANES,), 1, jnp.int32)
    sixteen = jnp.full((LANES,), 16, jnp.int32)
    himsk = jnp.full((LANES,), -65536, jnp.int32)
    ev_cols = jnp.full((LANES,), 2, jnp.int32) * lax.iota(jnp.int32, 16)

    def tile(g, carry):
        base = base0 + g * T
        rbase = (base // IW) * W
        pltpu.sync_copy(widx_hbm.at[cid, pl.ds(rbase, NR)], wflat_v)
        pltpu.sync_copy(recip_hbm.at[pl.ds(base, T)], recip_v)

        handles = [
            pltpu.async_copy(
                shard_sp.at[wflat_v.at[r]], gath_v.at[pl.ds(r * IW, IW)], dsem
            )
            for r in range(NR)
        ]
        for h in handles:
            h.wait()

        def item(t, c2):
            r = plsc.load_gather(recip_v, [jnp.full((LANES,), t, jnp.int32)])
            trow = jnp.full((LANES,), t, jnp.int32)
            for ch in range(2):
                acc = gath_v[t * W, pl.ds(ch * 32, 32)]
                for j in range(1, W):
                    acc = acc + gath_v[t * W + j, pl.ds(ch * 32, 32)]
                v = plsc.bitcast(acc, jnp.int32)
                even = plsc.bitcast(lax.shift_left(v, sixteen), jnp.float32) * r
                odd = plsc.bitcast(v & himsk, jnp.float32) * r
                base_c = jnp.full((LANES,), ch * 32, jnp.int32)
                plsc.store_scatter(outs_v, [trow, base_c + ev_cols], even)
                plsc.store_scatter(outs_v, [trow, base_c + ev_cols + one16], odd)
            return c2

        lax.fori_loop(0, T, item, 0, unroll=4)

        pltpu.sync_copy(outs_v, out_hbm.at[cid, pl.ds(base, T)])
        return carry

    lax.fori_loop(0, NTB, tile, 0, unroll=False)


def _combine(p_ref, o_ref):
    o_ref[...] = p_ref[0] + p_ref[1]


@jax.jit
def kernel(sememes, sememe_to_word, word_table):
    sem_flat = sememes.reshape(M)
    s2w_pad = jnp.concatenate(
        [sememe_to_word, jnp.zeros((VOCAB, WP - W), jnp.int32)], axis=1
    )
    row_ids = lax.broadcasted_iota(jnp.int32, (VOCAB, 1), 0)
    wtb = (word_table * (row_ids != 0)).astype(jnp.bfloat16)
    zpad = jnp.zeros((SHR - HV, E), jnp.bfloat16)
    sh0 = jnp.concatenate([wtb[:HV], zpad], axis=0)
    sh1 = jnp.concatenate(
        [jnp.zeros((1, E), jnp.bfloat16), wtb[HV:], zpad[1:]], axis=0
    )
    sh = jnp.stack([sh0, sh1])

    mesh = plsc.VectorSubcoreMesh(core_axis_name="c", subcore_axis_name="s")
    sc_params = pltpu.CompilerParams(
        needs_layout_passes=False, use_tc_tiling_on_sc=False
    )

    fa = pl.kernel(
        _phase_a,
        out_type=(
            jax.ShapeDtypeStruct((NC, M * W // IW, IW), jnp.int32),
            jax.ShapeDtypeStruct((M,), jnp.float32),
        ),
        scratch_types=[
            pltpu.VMEM((1, IW), jnp.int32),
            pltpu.VMEM((T, WP), jnp.int32),
            pltpu.VMEM((NC, NR, IW), jnp.int32),
            pltpu.VMEM((T,), jnp.float32),
            pltpu.VMEM_SHARED((VOCAB, WP), jnp.int32),
        ],
        mesh=mesh,
        compiler_params=sc_params,
    )
    widx, recip = fa(sem_flat, s2w_pad)

    fb = pl.kernel(
        _phase_b,
        out_type=jax.ShapeDtypeStruct((NC, M, E), jnp.float32),
        scratch_types=[
            pltpu.VMEM((NR, IW), jnp.int32),
            pltpu.VMEM((T * W, E), jnp.bfloat16),
            pltpu.VMEM((T, E), jnp.float32),
            pltpu.VMEM((T,), jnp.float32),
            pltpu.VMEM_SHARED((SHR, E), jnp.bfloat16),
            pltpu.SemaphoreType.DMA,
        ],
        mesh=mesh,
        compiler_params=sc_params,
    )
    partials = fb(widx, recip, sh)

    RB = 2048
    out = pl.pallas_call(
        _combine,
        out_shape=jax.ShapeDtypeStruct((M, E), jnp.float32),
        grid=(M // RB,),
        in_specs=[pl.BlockSpec((NC, RB, E), lambda i: (0, i, 0))],
        out_specs=pl.BlockSpec((RB, E), lambda i: (i, 0)),
    )(partials)
    return out.reshape(B, L, E)
